# baseline (speedup 1.0000x reference)
import functools

import jax
import jax.numpy as jnp
from jax import lax
from jax.experimental import pallas as pl
from jax.experimental.pallas import tpu as pltpu
from jax.experimental.pallas import tpu_sc as plsc

B = 16384
D = 64
C = 5
NW = 32
BPW = B // NW
CHUNK = 128
NCHUNK = BPW // CHUNK


def _unpack2(v32):
    return plsc.unpack(v32, format=plsc.PackFormat.INTERLEAVED,
                       preferred_element_type=jnp.float32)


def _w2v_body(ttab, ctab, tidx_hbm, cidx_hbm, out_hbm,
              tidx_v, cidx_v, trows_v, crows_v, out_v, sem):
    wid = lax.axis_index("s") * 2 + lax.axis_index("c")

    for chunk in range(NCHUNK):
        row = wid * NCHUNK + chunk
        base = row * CHUNK

        pltpu.sync_copy(tidx_hbm.at[pl.ds(base, CHUNK)], tidx_v)
        pltpu.sync_copy(cidx_hbm.at[pl.ds(base * C, CHUNK * C)], cidx_v)

        cps = [pltpu.async_copy(ttab.at[tidx_v], trows_v, sem)]
        for j in range(C):
            cps.append(pltpu.async_copy(
                ctab.at[cidx_v.at[pl.ds(j * CHUNK, CHUNK)]],
                crows_v.at[pl.ds(j * CHUNK, CHUNK)], sem))
        for cp in cps:
            cp.wait()

        lane15 = jnp.arange(16, dtype=jnp.int32) == 15

        def body(g, _):
            for bl in range(16):
                i = g * 16 + bl
                t = []
                for k in range(2):
                    t.extend(_unpack2(trows_v[i, pl.ds(32 * k, 32)]))
                for c in range(C):
                    r = i * C + c
                    acc = None
                    for k in range(2):
                        c0, c1 = _unpack2(crows_v[r, pl.ds(32 * k, 32)])
                        p = t[2 * k] * c0 + t[2 * k + 1] * c1
                        acc = p if acc is None else acc + p
                    cums = jnp.cumsum(acc)
                    plsc.store_scatter(
                        out_v, [jnp.full((16,), r, jnp.int32)], cums,
                        mask=lane15)
            return 0

        lax.fori_loop(0, CHUNK // 16, body, 0)

        pltpu.sync_copy(out_v, out_hbm.at[pl.ds(base * C, CHUNK * C)])


def kernel(target, context, target_table, context_table):
    mesh = plsc.VectorSubcoreMesh(core_axis_name="c", subcore_axis_name="s")
    ctx_flat = context.reshape(B * C).astype(jnp.int32)
    tgt = target.astype(jnp.int32)
    ttab = target_table.astype(jnp.bfloat16)
    ctab = context_table.astype(jnp.bfloat16)

    run = functools.partial(
        pl.kernel,
        mesh=mesh,
        compiler_params=pltpu.CompilerParams(
            needs_layout_passes=False, use_tc_tiling_on_sc=False),
        out_type=jax.ShapeDtypeStruct((B * C,), jnp.float32),
        scratch_types=[
            pltpu.VMEM((CHUNK,), jnp.int32),
            pltpu.VMEM((CHUNK * C,), jnp.int32),
            pltpu.VMEM((CHUNK, D), jnp.bfloat16),
            pltpu.VMEM((CHUNK * C, D), jnp.bfloat16),
            pltpu.VMEM((CHUNK * C,), jnp.float32),
            pltpu.SemaphoreType.DMA,
        ],
    )(_w2v_body)

    out = run(ttab, ctab, tgt, ctx_flat)
    return out.reshape(B, C)


# speedup vs baseline: 1.0178x; 1.0178x over previous
import functools

import jax
import jax.numpy as jnp
from jax import lax
from jax.experimental import pallas as pl
from jax.experimental.pallas import tpu as pltpu
from jax.experimental.pallas import tpu_sc as plsc

B = 16384
D = 64
C = 5
VOC = 1000000
NW = 32
W = 512
NBLK = VOC // W + 1
NBLKP = 1984
N_T = B
N_C = B * C
PAD = 128

_I16 = lambda x: jnp.full((16,), x, jnp.int32)


def _iota16():
    return jnp.arange(16, dtype=jnp.int32)


def _bin_body(tidx, cidx, pv_t, pp_t, pv_c, pp_c, off_t, off_c,
              ibuf, sbuf, hist_g, hist_e, off_v, base_v, slot2, pos2, sem):
    wid = lax.axis_index("s") * 2 + lax.axis_index("c")
    iota = _iota16()
    ones = _I16(1)
    zeros16 = jnp.zeros((16,), jnp.int32)
    masks = [iota == l for l in range(16)]

    for idx_hbm, n, pv_out, pp_out, off_out in (
            (tidx, N_T, pv_t, pp_t, off_t),
            (cidx, N_C, pv_c, pp_c, off_c)):
        sl = n // NW
        my_lo = wid * sl

        for k in range(NBLKP // 16):
            hist_g[pl.ds(k * 16, 16)] = zeros16
            hist_e[pl.ds(k * 16, 16)] = zeros16

        def hch(ch, _):
            pltpu.sync_copy(idx_hbm.at[pl.ds(ch * 2048, 2048)], ibuf)

            def hj(j, _):
                v = ibuf[pl.ds(pl.multiple_of(16 * j, 16), 16)]
                blk = v >> 9
                plsc.addupdate_scatter(hist_g, [blk], ones)
                pv = _I16(ch * 2048 + 16 * j) + iota
                plsc.addupdate_scatter(hist_e, [blk], ones,
                                       mask=pv < _I16(my_lo))
                return 0

            lax.fori_loop(0, 128, hj, 0)
            return 0

        lax.fori_loop(0, n // 2048, hch, 0)

        lane15 = iota == 15

        def pfx(k, carry):
            k16 = pl.multiple_of(16 * k, 16)
            h = hist_g[pl.ds(k16, 16)]
            cs = jnp.cumsum(h)
            off_v[pl.ds(k16, 16)] = cs - h + carry
            base_v[pl.ds(k16, 16)] = cs - h + carry + hist_e[pl.ds(k16, 16)]
            return carry + jnp.sum(jnp.where(lane15, cs, 0))

        lax.fori_loop(0, NBLKP // 16, pfx, jnp.int32(0))

        @pl.when(wid == 0)
        def _():
            pltpu.sync_copy(off_v, off_out)

        pltpu.sync_copy(idx_hbm.at[pl.ds(my_lo, sl)], sbuf.at[pl.ds(0, sl)])

        def slotch(j, _):
            v = sbuf[pl.ds(16 * j, 16)]
            blk = v >> 9
            slot = zeros16
            for l in range(16):
                g = plsc.load_gather(base_v, [blk], mask=masks[l])
                slot = jnp.where(masks[l], g, slot)
                plsc.addupdate_scatter(base_v, [blk], ones, mask=masks[l])
            r, q = j // 8, j % 8
            slot2[r, pl.ds(16 * q, 16)] = slot
            pos2[r, pl.ds(16 * q, 16)] = _I16(my_lo + 16 * j) + iota
            return 0

        lax.fori_loop(0, sl // 16, slotch, 0)

        cps = []
        for ch in range(sl // 128):
            cps.append(pltpu.async_copy(
                sbuf.at[pl.ds(ch * 128, 128)],
                pv_out.at[slot2.at[ch]], sem))
            cps.append(pltpu.async_copy(
                pos2.at[ch], pp_out.at[slot2.at[ch]], sem))
        for cp in cps:
            cp.wait()


def _sweep_body(ttab_t, ctab_t, pv_t, pp_t, pv_c, pp_c, off_t, off_c,
                gat_t, gat_c, buf0, buf1, tailb, stage, off_v, vbuf, pbuf,
                sem, sem2):
    wid = lax.axis_index("s") * 2 + lax.axis_index("c")
    iota = _iota16()
    cnt = jnp.where(wid < 2, 62, jnp.where(wid == 31, 60, 61))
    lo = 61 * wid + jnp.minimum(wid, 2)
    bufs = [buf0, buf1]

    for tab, pv_in, pp_in, off_in, gat, ndump in (
            (ttab_t, pv_t, pp_t, off_t, gat_t, N_T),
            (ctab_t, pv_c, pp_c, off_c, gat_c, N_C)):
        pltpu.sync_copy(off_in, off_v)
        dump = ndump + wid

        def blk_dma(j, slot):
            pltpu.async_copy(tab.at[:, pl.ds(j * W, W)], bufs[slot], sem)

        def blk_wait(slot):
            pltpu.make_async_copy(tab.at[:, pl.ds(0, W)],
                                  bufs[slot], sem).wait()

        def process(j, buf):
            j8 = j & ~jnp.int32(7)
            ov = off_v[pl.ds(pl.multiple_of(j8, 8), 16)]
            l0 = j - j8
            s0 = jnp.sum(jnp.where(iota == _I16(l0), ov, 0))
            s1 = jnp.sum(jnp.where(iota == _I16(l0 + 1), ov, 0))
            a = s0 & ~jnp.int32(7)
            trips = (s1 - a + 127) >> 7

            def chunk(k, _):
                q0 = pl.multiple_of(a + 128 * k, 8)
                pltpu.sync_copy(pv_in.at[pl.ds(q0, 128)], vbuf)
                pltpu.sync_copy(pp_in.at[pl.ds(q0, 128)], pbuf)

                def rbody(r, _):
                    r16 = pl.multiple_of(16 * r, 16)
                    q = _I16(q0 + 16 * r) + iota
                    m = (q >= _I16(s0)) & (q < _I16(s1))
                    v = vbuf[pl.ds(r16, 16)]
                    col = v - _I16(j * W)
                    row16 = _I16(16 * r) + iota
                    for d in range(D):
                        val = plsc.load_gather(
                            buf, [_I16(d), col], mask=m)
                        plsc.store_scatter(
                            stage, [row16, _I16(d)], val, mask=m)
                    pb = pbuf[pl.ds(r16, 16)]
                    pbuf[pl.ds(r16, 16)] = jnp.where(m, pb, _I16(dump))
                    return 0

                lax.fori_loop(0, 8, rbody, 0)
                pltpu.async_copy(stage, gat.at[pbuf], sem2).wait()
                return 0

            lax.fori_loop(0, trips, chunk, 0)

        blk_dma(lo, 0)

        @pl.when(cnt > 1)
        def _():
            blk_dma(lo + 1, 1)

        def pair(g, _):
            for b in range(2):
                j = 2 * g + b

                @pl.when(j < cnt)
                def _():
                    blk_wait(b)
                    process(lo + j, bufs[b])

                    @pl.when(j + 2 < cnt)
                    def _():
                        blk_dma(lo + j + 2, b)
            return 0

        lax.fori_loop(0, 31, pair, 0)

        @pl.when(wid == 31)
        def _():
            pltpu.sync_copy(tab.at[:, pl.ds((NBLK - 1) * W, 64)], tailb)
            process(jnp.int32(NBLK - 1), tailb)


def _dots_body(gat_t, gat_c, out_hbm, trows_v, crows_v, out_v, sem):
    wid = lax.axis_index("s") * 2 + lax.axis_index("c")
    iota = _iota16()
    lane15 = iota == 15

    for chunk in range(4):
        base = (wid * 4 + chunk) * 128
        cp1 = pltpu.async_copy(
            gat_t.at[pl.ds(base, 128)], trows_v, sem)
        cp2 = pltpu.async_copy(
            gat_c.at[pl.ds(base * C, 128 * C)], crows_v, sem)
        cp1.wait()
        cp2.wait()

        def body(g, _):
            for bl in range(16):
                i = g * 16 + bl
                t = [trows_v[i, pl.ds(16 * k, 16)] for k in range(4)]
                for c in range(C):
                    r = i * C + c
                    acc = t[0] * crows_v[r, pl.ds(0, 16)]
                    for k in range(1, 4):
                        acc = acc + t[k] * crows_v[r, pl.ds(16 * k, 16)]
                    cums = jnp.cumsum(acc)
                    plsc.store_scatter(out_v, [_I16(r)], cums, mask=lane15)
            return 0

        lax.fori_loop(0, 8, body, 0)
        pltpu.sync_copy(out_v, out_hbm.at[pl.ds(base * C, 128 * C)])


def kernel(target, context, target_table, context_table):
    mesh = plsc.VectorSubcoreMesh(core_axis_name="c", subcore_axis_name="s")
    ctx_flat = context.reshape(N_C).astype(jnp.int32)
    tgt = target.astype(jnp.int32)

    i32 = jnp.int32
    binned = functools.partial(
        pl.kernel, mesh=mesh,
        compiler_params=pltpu.CompilerParams(
            needs_layout_passes=False, use_tc_tiling_on_sc=False),
        out_type=(
            jax.ShapeDtypeStruct((N_T + PAD,), i32),
            jax.ShapeDtypeStruct((N_T + PAD,), i32),
            jax.ShapeDtypeStruct((N_C + PAD,), i32),
            jax.ShapeDtypeStruct((N_C + PAD,), i32),
            jax.ShapeDtypeStruct((NBLKP,), i32),
            jax.ShapeDtypeStruct((NBLKP,), i32),
        ),
        scratch_types=[
            pltpu.VMEM((2048,), i32),
            pltpu.VMEM((N_C // NW,), i32),
            pltpu.VMEM((NBLKP,), i32),
            pltpu.VMEM((NBLKP,), i32),
            pltpu.VMEM((NBLKP,), i32),
            pltpu.VMEM((NBLKP,), i32),
            pltpu.VMEM((N_C // NW // 128, 128), i32),
            pltpu.VMEM((N_C // NW // 128, 128), i32),
            pltpu.SemaphoreType.DMA,
        ],
    )(_bin_body)

    pv_t, pp_t, pv_c, pp_c, off_t, off_c = binned(tgt, ctx_flat)

    swept = functools.partial(
        pl.kernel, mesh=mesh,
        compiler_params=pltpu.CompilerParams(
            needs_layout_passes=False, use_tc_tiling_on_sc=True),
        out_type=(
            jax.ShapeDtypeStruct((N_T + PAD, 128), jnp.float32),
            jax.ShapeDtypeStruct((N_C + PAD, 128), jnp.float32),
        ),
        scratch_types=[
            pltpu.VMEM((D, W), jnp.float32),
            pltpu.VMEM((D, W), jnp.float32),
            pltpu.VMEM((D, 64), jnp.float32),
            pltpu.VMEM((128, 128), jnp.float32),
            pltpu.VMEM((NBLKP,), i32),
            pltpu.VMEM((128,), i32),
            pltpu.VMEM((128,), i32),
            pltpu.SemaphoreType.DMA,
            pltpu.SemaphoreType.DMA,
        ],
    )(_sweep_body)

    gat_t, gat_c = swept(target_table.T, context_table.T,
                         pv_t, pp_t, pv_c, pp_c, off_t, off_c)

    dots = functools.partial(
        pl.kernel, mesh=mesh,
        compiler_params=pltpu.CompilerParams(
            needs_layout_passes=False, use_tc_tiling_on_sc=True),
        out_type=jax.ShapeDtypeStruct((N_C,), jnp.float32),
        scratch_types=[
            pltpu.VMEM((128, 128), jnp.float32),
            pltpu.VMEM((128 * C, 128), jnp.float32),
            pltpu.VMEM((128 * C,), jnp.float32),
            pltpu.SemaphoreType.DMA,
        ],
    )(_dots_body)

    out = dots(gat_t, gat_c)
    return out.reshape(B, C)


# speedup vs baseline: 1.0652x; 1.0465x over previous
import functools

import jax
import jax.numpy as jnp
from jax import lax
from jax.experimental import pallas as pl
from jax.experimental.pallas import tpu as pltpu
from jax.experimental.pallas import tpu_sc as plsc

B = 16384
D = 64
C = 5
VOC = 1000000
NW = 32
W = 512
NBLK = VOC // W + 1
NBLKP = 1984
N_T = B
N_C = B * C
PAD = 128

_I16 = lambda x: jnp.full((16,), x, jnp.int32)


def _iota16():
    return jnp.arange(16, dtype=jnp.int32)


def _bin_body(tidx, cidx, pv_t, pp_t, pv_c, pp_c, off_t, off_c,
              ibuf, sbuf, hg0, hg1, hg2, hg3, he0, he1, he2, he3,
              off_v, base_v, slot2, pos2, sem):
    hgs = [hg0, hg1, hg2, hg3]
    hes = [he0, he1, he2, he3]
    wid = lax.axis_index("s") * 2 + lax.axis_index("c")
    iota = _iota16()
    ones = _I16(1)
    zeros16 = jnp.zeros((16,), jnp.int32)
    masks = [iota == l for l in range(16)]

    for idx_hbm, n, pv_out, pp_out, off_out in (
            (tidx, N_T, pv_t, pp_t, off_t),
            (cidx, N_C, pv_c, pp_c, off_c)):
        sl = n // NW
        my_lo = wid * sl

        def zk(k, _):
            k16 = pl.multiple_of(16 * k, 16)
            for h in hgs + hes:
                h[pl.ds(k16, 16)] = zeros16
            return 0

        lax.fori_loop(0, NBLKP // 16, zk, 0)

        def hch(ch, _):
            pltpu.sync_copy(idx_hbm.at[pl.ds(ch * 2048, 2048)], ibuf)

            def hj(j, _):
                for u in range(4):
                    o = pl.multiple_of(64 * j + 16 * u, 16)
                    v = ibuf[pl.ds(o, 16)]
                    blk = v >> 9
                    plsc.addupdate_scatter(hgs[u], [blk], ones)
                    pv = _I16(ch * 2048 + 64 * j + 16 * u) + iota
                    plsc.addupdate_scatter(hes[u], [blk], ones,
                                           mask=pv < _I16(my_lo))
                return 0

            lax.fori_loop(0, 32, hj, 0)
            return 0

        lax.fori_loop(0, n // 2048, hch, 0)

        lane15 = iota == 15

        def pfx(k, carry):
            k16 = pl.multiple_of(16 * k, 16)
            h = (hgs[0][pl.ds(k16, 16)] + hgs[1][pl.ds(k16, 16)]
                 + hgs[2][pl.ds(k16, 16)] + hgs[3][pl.ds(k16, 16)])
            he = (hes[0][pl.ds(k16, 16)] + hes[1][pl.ds(k16, 16)]
                  + hes[2][pl.ds(k16, 16)] + hes[3][pl.ds(k16, 16)])
            cs = jnp.cumsum(h)
            off_v[pl.ds(k16, 16)] = cs - h + carry
            base_v[pl.ds(k16, 16)] = cs - h + carry + he
            return carry + jnp.sum(jnp.where(lane15, cs, 0))

        lax.fori_loop(0, NBLKP // 16, pfx, jnp.int32(0))

        @pl.when(wid == 0)
        def _():
            pltpu.sync_copy(off_v, off_out)

        pltpu.sync_copy(idx_hbm.at[pl.ds(my_lo, sl)], sbuf.at[pl.ds(0, sl)])

        def slotch(j, _):
            v = sbuf[pl.ds(16 * j, 16)]
            blk = v >> 9
            slot = zeros16
            for l in range(16):
                g = plsc.load_gather(base_v, [blk], mask=masks[l])
                slot = jnp.where(masks[l], g, slot)
                plsc.addupdate_scatter(base_v, [blk], ones, mask=masks[l])
            r, q = j // 8, j % 8
            slot2[r, pl.ds(16 * q, 16)] = slot
            pos2[r, pl.ds(16 * q, 16)] = _I16(my_lo + 16 * j) + iota
            return 0

        lax.fori_loop(0, sl // 16, slotch, 0)

        cps = []
        for ch in range(sl // 128):
            cps.append(pltpu.async_copy(
                sbuf.at[pl.ds(ch * 128, 128)],
                pv_out.at[slot2.at[ch]], sem))
            cps.append(pltpu.async_copy(
                pos2.at[ch], pp_out.at[slot2.at[ch]], sem))
        for cp in cps:
            cp.wait()


def _sweep_body(ttab_t, ctab_t, pv_t, pp_t, pv_c, pp_c, off_t, off_c,
                gat_t, gat_c, buf0, buf1, tailb, stage,
                off_v, vb0, vb1, pb0, pb1, sidx, sem, sem2, sem3):
    wid = lax.axis_index("s") * 2 + lax.axis_index("c")
    iota = _iota16()
    cnt = jnp.where(wid < 2, 62, jnp.where(wid == 31, 60, 61))
    lo = 61 * wid + jnp.minimum(wid, 2)
    bufs = [buf0, buf1]
    vbs = [vb0, vb1]
    pbs = [pb0, pb1]
    stage_bytes = 128 * 128 * 4

    del stage_bytes
    def initdump(r, _):
        r16 = pl.multiple_of(16 * r, 16)
        sidx[pl.ds(r16, 16)] = _I16(N_T + wid)
        return 0

    lax.fori_loop(0, 8, initdump, 0)
    pltpu.async_copy(stage, gat_t.at[sidx], sem2)

    for tab, pv_in, pp_in, off_in, gat, ndump in (
            (ttab_t, pv_t, pp_t, off_t, gat_t, N_T),
            (ctab_t, pv_c, pp_c, off_c, gat_c, N_C)):
        pltpu.sync_copy(off_in, off_v)
        dump = ndump + wid

        def blk_dma(j, slot):
            pltpu.async_copy(tab.at[:, pl.ds(j * W, W)], bufs[slot], sem)

        def blk_wait(slot):
            pltpu.make_async_copy(tab.at[:, pl.ds(0, W)],
                                  bufs[slot], sem).wait()

        def bounds(j):
            j8 = j & ~jnp.int32(7)
            ov = off_v[pl.ds(pl.multiple_of(j8, 8), 16)]
            l0 = j - j8
            s0 = jnp.sum(jnp.where(iota == _I16(l0), ov, 0))
            s1 = jnp.sum(jnp.where(iota == _I16(l0 + 1), ov, 0))
            return s0, s1

        def idx_dma(j, slot):
            s0, _ = bounds(j)
            q0 = pl.multiple_of(s0 & ~jnp.int32(7), 8)
            pltpu.async_copy(pv_in.at[pl.ds(q0, 128)], vbs[slot], sem3)
            pltpu.async_copy(pp_in.at[pl.ds(q0, 128)], pbs[slot], sem3)

        def idx_wait(slot):
            pltpu.make_async_copy(pv_in.at[pl.ds(0, 128)],
                                  vbs[slot], sem3).wait()
            pltpu.make_async_copy(pp_in.at[pl.ds(0, 128)],
                                  pbs[slot], sem3).wait()

        def process(j, buf, vbuf, pbuf):
            s0, s1 = bounds(j)
            a = s0 & ~jnp.int32(7)
            trips = (s1 - a + 127) >> 7

            def chunk(k, _):
                q0 = pl.multiple_of(a + 128 * k, 8)

                @pl.when(k > 0)
                def _():
                    pltpu.sync_copy(pv_in.at[pl.ds(q0, 128)], vbuf)
                    pltpu.sync_copy(pp_in.at[pl.ds(q0, 128)], pbuf)

                pltpu.make_async_copy(stage, gat.at[sidx], sem2).wait()

                def rbody(r, _):
                    r16 = pl.multiple_of(16 * r, 16)
                    q = _I16(q0 + 16 * r) + iota
                    m = (q >= _I16(s0)) & (q < _I16(s1))
                    v = vbuf[pl.ds(r16, 16)]
                    col = v - _I16(j * W)
                    row16 = _I16(16 * r) + iota
                    for d in range(D):
                        val = plsc.load_gather(
                            buf, [_I16(d), col], mask=m)
                        plsc.store_scatter(
                            stage, [row16, _I16(d)], val, mask=m)
                    pb = pbuf[pl.ds(r16, 16)]
                    pbuf[pl.ds(r16, 16)] = jnp.where(m, pb, _I16(dump))
                    return 0

                lax.fori_loop(0, 8, rbody, 0)

                def cpb(r, _):
                    r16 = pl.multiple_of(16 * r, 16)
                    sidx[pl.ds(r16, 16)] = pbuf[pl.ds(r16, 16)]
                    return 0

                lax.fori_loop(0, 8, cpb, 0)
                pltpu.async_copy(stage, gat.at[sidx], sem2)
                return 0

            lax.fori_loop(0, trips, chunk, 0)

        blk_dma(lo, 0)
        idx_dma(lo, 0)

        @pl.when(cnt > 1)
        def _():
            blk_dma(lo + 1, 1)
            idx_dma(lo + 1, 1)

        def pair(g, _):
            for b in range(2):
                j = 2 * g + b

                @pl.when(j < cnt)
                def _():
                    blk_wait(b)
                    idx_wait(b)
                    process(lo + j, bufs[b], vbs[b], pbs[b])

                    @pl.when(j + 2 < cnt)
                    def _():
                        blk_dma(lo + j + 2, b)
                        idx_dma(lo + j + 2, b)
            return 0

        lax.fori_loop(0, 31, pair, 0)

        @pl.when(wid == 31)
        def _():
            pltpu.sync_copy(tab.at[:, pl.ds((NBLK - 1) * W, 64)], tailb)
            idx_dma(jnp.int32(NBLK - 1), 0)
            idx_wait(0)
            process(jnp.int32(NBLK - 1), tailb, vbs[0], pbs[0])

    pltpu.make_async_copy(stage, gat_c.at[sidx], sem2).wait()


def _dots_body(gat_t, gat_c, out_hbm, trows_v, crows_v, out_v, sem):
    wid = lax.axis_index("s") * 2 + lax.axis_index("c")
    iota = _iota16()
    lane15 = iota == 15

    for chunk in range(4):
        base = (wid * 4 + chunk) * 128
        cp1 = pltpu.async_copy(
            gat_t.at[pl.ds(base, 128)], trows_v, sem)
        cp2 = pltpu.async_copy(
            gat_c.at[pl.ds(base * C, 128 * C)], crows_v, sem)
        cp1.wait()
        cp2.wait()

        def body(g, _):
            for bl in range(16):
                i = g * 16 + bl
                t = [trows_v[i, pl.ds(16 * k, 16)] for k in range(4)]
                for c in range(C):
                    r = i * C + c
                    acc = t[0] * crows_v[r, pl.ds(0, 16)]
                    for k in range(1, 4):
                        acc = acc + t[k] * crows_v[r, pl.ds(16 * k, 16)]
                    cums = jnp.cumsum(acc)
                    plsc.store_scatter(out_v, [_I16(r)], cums, mask=lane15)
            return 0

        lax.fori_loop(0, 8, body, 0)
        pltpu.sync_copy(out_v, out_hbm.at[pl.ds(base * C, 128 * C)])


def kernel(target, context, target_table, context_table):
    mesh = plsc.VectorSubcoreMesh(core_axis_name="c", subcore_axis_name="s")
    ctx_flat = context.reshape(N_C).astype(jnp.int32)
    tgt = target.astype(jnp.int32)

    i32 = jnp.int32
    binned = functools.partial(
        pl.kernel, mesh=mesh,
        compiler_params=pltpu.CompilerParams(
            needs_layout_passes=False, use_tc_tiling_on_sc=False),
        out_type=(
            jax.ShapeDtypeStruct((N_T + PAD,), i32),
            jax.ShapeDtypeStruct((N_T + PAD,), i32),
            jax.ShapeDtypeStruct((N_C + PAD,), i32),
            jax.ShapeDtypeStruct((N_C + PAD,), i32),
            jax.ShapeDtypeStruct((NBLKP,), i32),
            jax.ShapeDtypeStruct((NBLKP,), i32),
        ),
        scratch_types=(
            [pltpu.VMEM((2048,), i32),
             pltpu.VMEM((N_C // NW,), i32)]
            + [pltpu.VMEM((NBLKP,), i32)] * 8
            + [pltpu.VMEM((NBLKP,), i32),
               pltpu.VMEM((NBLKP,), i32),
               pltpu.VMEM((N_C // NW // 128, 128), i32),
               pltpu.VMEM((N_C // NW // 128, 128), i32),
               pltpu.SemaphoreType.DMA]
        ),
    )(_bin_body)

    pv_t, pp_t, pv_c, pp_c, off_t, off_c = binned(tgt, ctx_flat)

    swept = functools.partial(
        pl.kernel, mesh=mesh,
        compiler_params=pltpu.CompilerParams(
            needs_layout_passes=False, use_tc_tiling_on_sc=True),
        out_type=(
            jax.ShapeDtypeStruct((N_T + PAD, 128), jnp.float32),
            jax.ShapeDtypeStruct((N_C + PAD, 128), jnp.float32),
        ),
        scratch_types=[
            pltpu.VMEM((D, W), jnp.float32),
            pltpu.VMEM((D, W), jnp.float32),
            pltpu.VMEM((D, 64), jnp.float32),
            pltpu.VMEM((128, 128), jnp.float32),
            pltpu.VMEM((NBLKP,), i32),
            pltpu.VMEM((128,), i32),
            pltpu.VMEM((128,), i32),
            pltpu.VMEM((128,), i32),
            pltpu.VMEM((128,), i32),
            pltpu.VMEM((128,), i32),
            pltpu.SemaphoreType.DMA,
            pltpu.SemaphoreType.DMA,
            pltpu.SemaphoreType.DMA,
        ],
    )(_sweep_body)

    gat_t, gat_c = swept(target_table.T, context_table.T,
                         pv_t, pp_t, pv_c, pp_c, off_t, off_c)

    dots = functools.partial(
        pl.kernel, mesh=mesh,
        compiler_params=pltpu.CompilerParams(
            needs_layout_passes=False, use_tc_tiling_on_sc=True),
        out_type=jax.ShapeDtypeStruct((N_C,), jnp.float32),
        scratch_types=[
            pltpu.VMEM((128, 128), jnp.float32),
            pltpu.VMEM((128 * C, 128), jnp.float32),
            pltpu.VMEM((128 * C,), jnp.float32),
            pltpu.SemaphoreType.DMA,
        ],
    )(_dots_body)

    out = dots(gat_t, gat_c)
    return out.reshape(B, C)


# speedup vs baseline: 1.3338x; 1.2522x over previous
import functools

import jax
import jax.numpy as jnp
from jax import lax
from jax.experimental import pallas as pl
from jax.experimental.pallas import tpu as pltpu
from jax.experimental.pallas import tpu_sc as plsc

B = 16384
D = 64
C = 5
VOC = 1000000
NW = 32
W = 512
NBLK = VOC // W + 1
NBLKP = 1984
N_T = B
N_C = B * C
PAD = 128

_I16 = lambda x: jnp.full((16,), x, jnp.int32)


def _iota16():
    return jnp.arange(16, dtype=jnp.int32)


def _bin_body(tidx, cidx, pv_t, pp_t, pv_c, pp_c, off_t, off_c,
              ibuf, sbuf, hg0, hg1, hg2, hg3, he0, he1, he2, he3,
              off_v, base_v, slot2, pos2, sem):
    hgs = [hg0, hg1, hg2, hg3]
    hes = [he0, he1, he2, he3]
    wid = lax.axis_index("s") * 2 + lax.axis_index("c")
    iota = _iota16()
    ones = _I16(1)
    zeros16 = jnp.zeros((16,), jnp.int32)
    masks = [iota == l for l in range(16)]

    for idx_hbm, n, pv_out, pp_out, off_out in (
            (tidx, N_T, pv_t, pp_t, off_t),
            (cidx, N_C, pv_c, pp_c, off_c)):
        sl = n // NW
        my_lo = wid * sl

        def zk(k, _):
            k16 = pl.multiple_of(16 * k, 16)
            for h in hgs + hes:
                h[pl.ds(k16, 16)] = zeros16
            return 0

        lax.fori_loop(0, NBLKP // 16, zk, 0)

        def ich_dma(ch, half):
            pltpu.async_copy(
                idx_hbm.at[pl.ds(pl.multiple_of(ch * 2048, 8), 2048)],
                ibuf.at[pl.ds(2048 * half, 2048)], sem)

        def ich_wait(half):
            pltpu.make_async_copy(idx_hbm.at[pl.ds(0, 2048)],
                                  ibuf.at[pl.ds(2048 * half, 2048)],
                                  sem).wait()

        ich_dma(0, 0)
        if n // 2048 > 1:
            ich_dma(1, 1)

        def hch(g, _):
            for h in range(2):
                ch = 2 * g + h

                @pl.when(ch < n // 2048)
                def _():
                    ich_wait(h)

                    def hj(j, _):
                        for u in range(4):
                            o = pl.multiple_of(2048 * h + 64 * j + 16 * u, 16)
                            v = ibuf[pl.ds(o, 16)]
                            blk = v >> 9
                            plsc.addupdate_scatter(hgs[u], [blk], ones)
                            pv = _I16(ch * 2048 + 64 * j + 16 * u) + iota
                            plsc.addupdate_scatter(hes[u], [blk], ones,
                                                   mask=pv < _I16(my_lo))
                        return 0

                    lax.fori_loop(0, 32, hj, 0)

                    @pl.when(ch + 2 < n // 2048)
                    def _():
                        ich_dma(ch + 2, h)
            return 0

        lax.fori_loop(0, (n // 2048 + 1) // 2, hch, 0)

        lane15 = iota == 15

        def pfx(k, carry):
            k16 = pl.multiple_of(16 * k, 16)
            h = (hgs[0][pl.ds(k16, 16)] + hgs[1][pl.ds(k16, 16)]
                 + hgs[2][pl.ds(k16, 16)] + hgs[3][pl.ds(k16, 16)])
            he = (hes[0][pl.ds(k16, 16)] + hes[1][pl.ds(k16, 16)]
                  + hes[2][pl.ds(k16, 16)] + hes[3][pl.ds(k16, 16)])
            cs = jnp.cumsum(h)
            off_v[pl.ds(k16, 16)] = cs - h + carry
            base_v[pl.ds(k16, 16)] = cs - h + carry + he
            return carry + jnp.sum(jnp.where(lane15, cs, 0))

        lax.fori_loop(0, NBLKP // 16, pfx, jnp.int32(0))

        @pl.when(wid == 0)
        def _():
            pltpu.sync_copy(off_v, off_out)

        pltpu.sync_copy(idx_hbm.at[pl.ds(my_lo, sl)], sbuf.at[pl.ds(0, sl)])

        def slotch(j, _):
            v = sbuf[pl.ds(16 * j, 16)]
            blk = v >> 9
            slot = zeros16
            for l in range(16):
                g = plsc.load_gather(base_v, [blk], mask=masks[l])
                slot = jnp.where(masks[l], g, slot)
                plsc.addupdate_scatter(base_v, [blk], ones, mask=masks[l])
            r, q = j // 8, j % 8
            slot2[r, pl.ds(16 * q, 16)] = slot
            pos2[r, pl.ds(16 * q, 16)] = _I16(my_lo + 16 * j) + iota
            return 0

        lax.fori_loop(0, sl // 16, slotch, 0)

        cps = []
        for ch in range(sl // 128):
            cps.append(pltpu.async_copy(
                sbuf.at[pl.ds(ch * 128, 128)],
                pv_out.at[slot2.at[ch]], sem))
            cps.append(pltpu.async_copy(
                pos2.at[ch], pp_out.at[slot2.at[ch]], sem))
        for cp in cps:
            cp.wait()


def _sweep_body(ttab_t, ctab_t, pv_t, pp_t, pv_c, pp_c, off_t, off_c,
                gat_t, gat_c, buf0, buf1, tailb, stage,
                off_v, vb0, vb1, pb0, pb1, sidx, sem, sem2, sem3):
    wid = lax.axis_index("s") * 2 + lax.axis_index("c")
    iota = _iota16()
    cnt = jnp.where(wid < 2, 62, jnp.where(wid == 31, 60, 61))
    lo = 61 * wid + jnp.minimum(wid, 2)
    bufs = [buf0, buf1]
    vbs = [vb0, vb1]
    pbs = [pb0, pb1]
    stage_bytes = 128 * 128 * 4

    del stage_bytes
    def initdump(r, _):
        r16 = pl.multiple_of(16 * r, 16)
        sidx[pl.ds(r16, 16)] = _I16(N_T + 16 * r) + _iota16()
        return 0

    lax.fori_loop(0, 8, initdump, 0)
    pltpu.async_copy(stage, gat_t.at[sidx], sem2)

    for tab, pv_in, pp_in, off_in, gat, ndump in (
            (ttab_t, pv_t, pp_t, off_t, gat_t, N_T),
            (ctab_t, pv_c, pp_c, off_c, gat_c, N_C)):
        pltpu.sync_copy(off_in, off_v)
        dump = ndump

        def blk_dma(j, slot):
            pltpu.async_copy(tab.at[:, pl.ds(j * W, W)], bufs[slot], sem)

        def blk_wait(slot):
            pltpu.make_async_copy(tab.at[:, pl.ds(0, W)],
                                  bufs[slot], sem).wait()

        def bounds(j):
            j8 = j & ~jnp.int32(7)
            ov = off_v[pl.ds(pl.multiple_of(j8, 8), 16)]
            l0 = j - j8
            s0 = jnp.sum(jnp.where(iota == _I16(l0), ov, 0))
            s1 = jnp.sum(jnp.where(iota == _I16(l0 + 1), ov, 0))
            return s0, s1

        def idx_dma(j, slot):
            s0, _ = bounds(j)
            q0 = pl.multiple_of(s0 & ~jnp.int32(7), 8)
            pltpu.async_copy(pv_in.at[pl.ds(q0, 128)], vbs[slot], sem3)
            pltpu.async_copy(pp_in.at[pl.ds(q0, 128)], pbs[slot], sem3)

        def idx_wait(slot):
            pltpu.make_async_copy(pv_in.at[pl.ds(0, 128)],
                                  vbs[slot], sem3).wait()
            pltpu.make_async_copy(pp_in.at[pl.ds(0, 128)],
                                  pbs[slot], sem3).wait()

        def process(j, buf, vbuf, pbuf):
            s0, s1 = bounds(j)
            a = s0 & ~jnp.int32(7)
            trips = (s1 - a + 127) >> 7

            def chunk(k, _):
                q0 = pl.multiple_of(a + 128 * k, 8)

                @pl.when(k > 0)
                def _():
                    pltpu.sync_copy(pv_in.at[pl.ds(q0, 128)], vbuf)
                    pltpu.sync_copy(pp_in.at[pl.ds(q0, 128)], pbuf)

                pltpu.make_async_copy(stage, gat.at[sidx], sem2).wait()

                def rbody(r, _):
                    r16 = pl.multiple_of(16 * r, 16)
                    q = _I16(q0 + 16 * r) + iota
                    m = (q >= _I16(s0)) & (q < _I16(s1))
                    v = vbuf[pl.ds(r16, 16)]
                    col = v - _I16(j * W)
                    row16 = _I16(16 * r) + iota
                    for d in range(D):
                        val = plsc.load_gather(
                            buf, [_I16(d), col], mask=m)
                        plsc.store_scatter(
                            stage, [row16, _I16(d)], val, mask=m)
                    pb = pbuf[pl.ds(r16, 16)]
                    spread = _I16(dump) + ((row16 + _I16(4 * wid)) & _I16(127))
                    pbuf[pl.ds(r16, 16)] = jnp.where(m, pb, spread)
                    return 0

                lax.fori_loop(0, 8, rbody, 0)

                def cpb(r, _):
                    r16 = pl.multiple_of(16 * r, 16)
                    sidx[pl.ds(r16, 16)] = pbuf[pl.ds(r16, 16)]
                    return 0

                lax.fori_loop(0, 8, cpb, 0)
                pltpu.async_copy(stage, gat.at[sidx], sem2)
                return 0

            lax.fori_loop(0, trips, chunk, 0)

        blk_dma(lo, 0)
        idx_dma(lo, 0)

        @pl.when(cnt > 1)
        def _():
            blk_dma(lo + 1, 1)
            idx_dma(lo + 1, 1)

        def pair(g, _):
            for b in range(2):
                j = 2 * g + b

                @pl.when(j < cnt)
                def _():
                    blk_wait(b)
                    idx_wait(b)
                    process(lo + j, bufs[b], vbs[b], pbs[b])

                    @pl.when(j + 2 < cnt)
                    def _():
                        blk_dma(lo + j + 2, b)
                        idx_dma(lo + j + 2, b)
            return 0

        lax.fori_loop(0, 31, pair, 0)

        @pl.when(wid == 31)
        def _():
            pltpu.sync_copy(tab.at[:, pl.ds((NBLK - 1) * W, 64)], tailb)
            idx_dma(jnp.int32(NBLK - 1), 0)
            idx_wait(0)
            process(jnp.int32(NBLK - 1), tailb, vbs[0], pbs[0])

    pltpu.make_async_copy(stage, gat_c.at[sidx], sem2).wait()


def _dots_body(gat_t, gat_c, out_hbm, trows_v, crows_v, out_v, sem):
    wid = lax.axis_index("s") * 2 + lax.axis_index("c")
    iota = _iota16()
    lane15 = iota == 15

    for chunk in range(4):
        base = (wid * 4 + chunk) * 128
        cp1 = pltpu.async_copy(
            gat_t.at[pl.ds(base, 128)], trows_v, sem)
        cp2 = pltpu.async_copy(
            gat_c.at[pl.ds(base * C, 128 * C)], crows_v, sem)
        cp1.wait()
        cp2.wait()

        def body(g, _):
            for bl in range(16):
                i = g * 16 + bl
                t = [trows_v[i, pl.ds(16 * k, 16)] for k in range(4)]
                for c in range(C):
                    r = i * C + c
                    acc = t[0] * crows_v[r, pl.ds(0, 16)]
                    for k in range(1, 4):
                        acc = acc + t[k] * crows_v[r, pl.ds(16 * k, 16)]
                    cums = jnp.cumsum(acc)
                    plsc.store_scatter(out_v, [_I16(r)], cums, mask=lane15)
            return 0

        lax.fori_loop(0, 8, body, 0)
        pltpu.sync_copy(out_v, out_hbm.at[pl.ds(base * C, 128 * C)])


def kernel(target, context, target_table, context_table):
    mesh = plsc.VectorSubcoreMesh(core_axis_name="c", subcore_axis_name="s")
    ctx_flat = context.reshape(N_C).astype(jnp.int32)
    tgt = target.astype(jnp.int32)

    i32 = jnp.int32
    binned = functools.partial(
        pl.kernel, mesh=mesh,
        compiler_params=pltpu.CompilerParams(
            needs_layout_passes=False, use_tc_tiling_on_sc=False),
        out_type=(
            jax.ShapeDtypeStruct((N_T + PAD,), i32),
            jax.ShapeDtypeStruct((N_T + PAD,), i32),
            jax.ShapeDtypeStruct((N_C + PAD,), i32),
            jax.ShapeDtypeStruct((N_C + PAD,), i32),
            jax.ShapeDtypeStruct((NBLKP,), i32),
            jax.ShapeDtypeStruct((NBLKP,), i32),
        ),
        scratch_types=(
            [pltpu.VMEM((4096,), i32),
             pltpu.VMEM((N_C // NW,), i32)]
            + [pltpu.VMEM((NBLKP,), i32)] * 8
            + [pltpu.VMEM((NBLKP,), i32),
               pltpu.VMEM((NBLKP,), i32),
               pltpu.VMEM((N_C // NW // 128, 128), i32),
               pltpu.VMEM((N_C // NW // 128, 128), i32),
               pltpu.SemaphoreType.DMA]
        ),
    )(_bin_body)

    pv_t, pp_t, pv_c, pp_c, off_t, off_c = binned(tgt, ctx_flat)

    swept = functools.partial(
        pl.kernel, mesh=mesh,
        compiler_params=pltpu.CompilerParams(
            needs_layout_passes=False, use_tc_tiling_on_sc=True),
        out_type=(
            jax.ShapeDtypeStruct((N_T + PAD, 128), jnp.float32),
            jax.ShapeDtypeStruct((N_C + PAD, 128), jnp.float32),
        ),
        scratch_types=[
            pltpu.VMEM((D, W), jnp.float32),
            pltpu.VMEM((D, W), jnp.float32),
            pltpu.VMEM((D, 64), jnp.float32),
            pltpu.VMEM((128, 128), jnp.float32),
            pltpu.VMEM((NBLKP,), i32),
            pltpu.VMEM((128,), i32),
            pltpu.VMEM((128,), i32),
            pltpu.VMEM((128,), i32),
            pltpu.VMEM((128,), i32),
            pltpu.VMEM((128,), i32),
            pltpu.SemaphoreType.DMA,
            pltpu.SemaphoreType.DMA,
            pltpu.SemaphoreType.DMA,
        ],
    )(_sweep_body)

    gat_t, gat_c = swept(target_table.T, context_table.T,
                         pv_t, pp_t, pv_c, pp_c, off_t, off_c)

    dots = functools.partial(
        pl.kernel, mesh=mesh,
        compiler_params=pltpu.CompilerParams(
            needs_layout_passes=False, use_tc_tiling_on_sc=True),
        out_type=jax.ShapeDtypeStruct((N_C,), jnp.float32),
        scratch_types=[
            pltpu.VMEM((128, 128), jnp.float32),
            pltpu.VMEM((128 * C, 128), jnp.float32),
            pltpu.VMEM((128 * C,), jnp.float32),
            pltpu.SemaphoreType.DMA,
        ],
    )(_dots_body)

    out = dots(gat_t, gat_c)
    return out.reshape(B, C)


# speedup vs baseline: 1.3807x; 1.0352x over previous
import functools

import jax
import jax.numpy as jnp
from jax import lax
from jax.experimental import pallas as pl
from jax.experimental.pallas import tpu as pltpu
from jax.experimental.pallas import tpu_sc as plsc

B = 16384
D = 64
C = 5
VOC = 1000000
NW = 32
W = 512
NBLK = VOC // W + 1
NBLKP = 1984
N_T = B
N_C = B * C
PAD = 128

_I16 = lambda x: jnp.full((16,), x, jnp.int32)


def _iota16():
    return jnp.arange(16, dtype=jnp.int32)


def _hist_body(tidx, cidx, hw_t, hw_c, sbuf, hg0, hg1, hg2, hg3, hsum):
    hgs = [hg0, hg1, hg2, hg3]
    wid = lax.axis_index("s") * 2 + lax.axis_index("c")
    ones = _I16(1)
    zeros16 = jnp.zeros((16,), jnp.int32)

    for idx_hbm, n, hw_out in ((tidx, N_T, hw_t), (cidx, N_C, hw_c)):
        sl = n // NW
        my_lo = wid * sl

        def zk(k, _):
            k16 = pl.multiple_of(16 * k, 16)
            for h in hgs:
                h[pl.ds(k16, 16)] = zeros16
            return 0

        lax.fori_loop(0, NBLKP // 16, zk, 0)
        pltpu.sync_copy(idx_hbm.at[pl.ds(my_lo, sl)], sbuf.at[pl.ds(0, sl)])

        def hj(j, _):
            for u in range(4):
                o = pl.multiple_of(64 * j + 16 * u, 16)
                v = sbuf[pl.ds(o, 16)]
                plsc.addupdate_scatter(hgs[u], [v >> 9], ones)
            return 0

        lax.fori_loop(0, sl // 64, hj, 0)

        def sk(k, _):
            k16 = pl.multiple_of(16 * k, 16)
            hsum[pl.ds(k16, 16)] = (
                hgs[0][pl.ds(k16, 16)] + hgs[1][pl.ds(k16, 16)]
                + hgs[2][pl.ds(k16, 16)] + hgs[3][pl.ds(k16, 16)])
            return 0

        lax.fori_loop(0, NBLKP // 16, sk, 0)
        pltpu.sync_copy(hsum, hw_out.at[wid])


def _bin_body(tidx, cidx, hw_t, hw_c, pv_t, pp_t, pv_c, pp_c, off_t, off_c,
              sbuf, hw_v, off_v, base_v, slot2, pos2, sem):
    wid = lax.axis_index("s") * 2 + lax.axis_index("c")
    iota = _iota16()
    ones = _I16(1)
    zeros16 = jnp.zeros((16,), jnp.int32)
    masks = [iota == l for l in range(16)]

    for idx_hbm, n, hw_in, pv_out, pp_out, off_out in (
            (tidx, N_T, hw_t, pv_t, pp_t, off_t),
            (cidx, N_C, hw_c, pv_c, pp_c, off_c)):
        sl = n // NW
        my_lo = wid * sl

        pltpu.sync_copy(hw_in, hw_v)

        lane15 = iota == 15

        def pfx(k, carry):
            k16 = pl.multiple_of(16 * k, 16)
            h = zeros16
            he = zeros16
            for u in range(NW):
                row = hw_v[u, pl.ds(k16, 16)]
                h = h + row
                he = he + jnp.where(u < wid, row, 0)
            cs = jnp.cumsum(h)
            off_v[pl.ds(k16, 16)] = cs - h + carry
            base_v[pl.ds(k16, 16)] = cs - h + carry + he
            return carry + jnp.sum(jnp.where(lane15, cs, 0))

        lax.fori_loop(0, NBLKP // 16, pfx, jnp.int32(0))

        @pl.when(wid == 0)
        def _():
            pltpu.sync_copy(off_v, off_out)

        pltpu.sync_copy(idx_hbm.at[pl.ds(my_lo, sl)], sbuf.at[pl.ds(0, sl)])

        def slotch(j, _):
            v = sbuf[pl.ds(16 * j, 16)]
            blk = v >> 9
            slot = zeros16
            for l in range(16):
                g = plsc.load_gather(base_v, [blk], mask=masks[l])
                slot = jnp.where(masks[l], g, slot)
                plsc.addupdate_scatter(base_v, [blk], ones, mask=masks[l])
            r, q = j // 8, j % 8
            slot2[r, pl.ds(16 * q, 16)] = slot
            pos2[r, pl.ds(16 * q, 16)] = _I16(my_lo + 16 * j) + iota
            return 0

        lax.fori_loop(0, sl // 16, slotch, 0)

        cps = []
        for ch in range(sl // 128):
            cps.append(pltpu.async_copy(
                sbuf.at[pl.ds(ch * 128, 128)],
                pv_out.at[slot2.at[ch]], sem))
            cps.append(pltpu.async_copy(
                pos2.at[ch], pp_out.at[slot2.at[ch]], sem))
        for cp in cps:
            cp.wait()


def _sweep_body(ttab_t, ctab_t, pv_t, pp_t, pv_c, pp_c, off_t, off_c,
                gat_t, gat_c, buf0, buf1, tailb, stage0, stage1,
                off_v, vb0, vb1, pb0, pb1, sidx0, sidx1,
                sem, sem2a, sem2b, sem3):
    wid = lax.axis_index("s") * 2 + lax.axis_index("c")
    iota = _iota16()
    cnt = jnp.where(wid < 2, 62, jnp.where(wid == 31, 60, 61))
    lo = 61 * wid + jnp.minimum(wid, 2)
    bufs = [buf0, buf1]
    vbs = [vb0, vb1]
    pbs = [pb0, pb1]
    stages = [stage0, stage1]
    sidxs = [sidx0, sidx1]
    sem2s = [sem2a, sem2b]

    def initdump(r, _):
        r16 = pl.multiple_of(16 * r, 16)
        sidx0[pl.ds(r16, 16)] = _I16(N_T + 16 * r) + _iota16()
        sidx1[pl.ds(r16, 16)] = _I16(N_T + 16 * r) + _iota16()
        return 0

    lax.fori_loop(0, 8, initdump, 0)
    pltpu.async_copy(stage0, gat_t.at[sidx0], sem2a)
    pltpu.async_copy(stage1, gat_t.at[sidx1], sem2b)

    for tab, pv_in, pp_in, off_in, gat, ndump in (
            (ttab_t, pv_t, pp_t, off_t, gat_t, N_T),
            (ctab_t, pv_c, pp_c, off_c, gat_c, N_C)):
        pltpu.sync_copy(off_in, off_v)
        dump = ndump

        def blk_dma(j, slot):
            pltpu.async_copy(tab.at[:, pl.ds(j * W, W)], bufs[slot], sem)

        def blk_wait(slot):
            pltpu.make_async_copy(tab.at[:, pl.ds(0, W)],
                                  bufs[slot], sem).wait()

        def bounds(j):
            j8 = j & ~jnp.int32(7)
            ov = off_v[pl.ds(pl.multiple_of(j8, 8), 16)]
            l0 = j - j8
            s0 = jnp.sum(jnp.where(iota == _I16(l0), ov, 0))
            s1 = jnp.sum(jnp.where(iota == _I16(l0 + 1), ov, 0))
            return s0, s1

        def idx_dma(j, slot):
            s0, _ = bounds(j)
            q0 = pl.multiple_of(s0 & ~jnp.int32(7), 8)
            pltpu.async_copy(pv_in.at[pl.ds(q0, 128)], vbs[slot], sem3)
            pltpu.async_copy(pp_in.at[pl.ds(q0, 128)], pbs[slot], sem3)

        def idx_wait(slot):
            pltpu.make_async_copy(pv_in.at[pl.ds(0, 128)],
                                  vbs[slot], sem3).wait()
            pltpu.make_async_copy(pp_in.at[pl.ds(0, 128)],
                                  pbs[slot], sem3).wait()

        def process(j, buf, vbuf, pbuf, stage, sidx, sem2):
            s0, s1 = bounds(j)
            a = s0 & ~jnp.int32(7)
            trips = (s1 - a + 127) >> 7

            def chunk(k, _):
                q0 = pl.multiple_of(a + 128 * k, 8)

                @pl.when(k > 0)
                def _():
                    pltpu.sync_copy(pv_in.at[pl.ds(q0, 128)], vbuf)
                    pltpu.sync_copy(pp_in.at[pl.ds(q0, 128)], pbuf)

                pltpu.make_async_copy(stage, gat.at[sidx], sem2).wait()

                def rbody(r, _):
                    r16 = pl.multiple_of(16 * r, 16)
                    q = _I16(q0 + 16 * r) + iota
                    m = (q >= _I16(s0)) & (q < _I16(s1))
                    v = vbuf[pl.ds(r16, 16)]
                    col = v - _I16(j * W)
                    row16 = _I16(16 * r) + iota
                    for d in range(D):
                        val = plsc.load_gather(
                            buf, [_I16(d), col], mask=m)
                        plsc.store_scatter(
                            stage, [row16, _I16(d)], val, mask=m)
                    pb = pbuf[pl.ds(r16, 16)]
                    spread = _I16(dump) + ((row16 + _I16(4 * wid)) & _I16(127))
                    pbuf[pl.ds(r16, 16)] = jnp.where(m, pb, spread)
                    return 0

                lax.fori_loop(0, 8, rbody, 0)

                def cpb(r, _):
                    r16 = pl.multiple_of(16 * r, 16)
                    sidx[pl.ds(r16, 16)] = pbuf[pl.ds(r16, 16)]
                    return 0

                lax.fori_loop(0, 8, cpb, 0)
                pltpu.async_copy(stage, gat.at[sidx], sem2)
                return 0

            lax.fori_loop(0, trips, chunk, 0)

        blk_dma(lo, 0)
        idx_dma(lo, 0)

        @pl.when(cnt > 1)
        def _():
            blk_dma(lo + 1, 1)
            idx_dma(lo + 1, 1)

        def pair(g, _):
            for b in range(2):
                j = 2 * g + b

                @pl.when(j < cnt)
                def _():
                    blk_wait(b)
                    idx_wait(b)
                    process(lo + j, bufs[b], vbs[b], pbs[b],
                            stages[b], sidxs[b], sem2s[b])

                    @pl.when(j + 2 < cnt)
                    def _():
                        blk_dma(lo + j + 2, b)
                        idx_dma(lo + j + 2, b)
            return 0

        lax.fori_loop(0, 31, pair, 0)

        @pl.when(wid == 31)
        def _():
            pltpu.sync_copy(tab.at[:, pl.ds((NBLK - 1) * W, 64)], tailb)
            idx_dma(jnp.int32(NBLK - 1), 0)
            idx_wait(0)
            process(jnp.int32(NBLK - 1), tailb, vbs[0], pbs[0],
                    stage0, sidx0, sem2a)

    pltpu.make_async_copy(stage0, gat_c.at[sidx0], sem2a).wait()
    pltpu.make_async_copy(stage1, gat_c.at[sidx1], sem2b).wait()


def _dots_body(gat_t, gat_c, out_hbm, trows_v, crows_v, out_v, sem):
    wid = lax.axis_index("s") * 2 + lax.axis_index("c")
    iota = _iota16()
    lane15 = iota == 15

    for chunk in range(4):
        base = (wid * 4 + chunk) * 128
        cp1 = pltpu.async_copy(
            gat_t.at[pl.ds(base, 128)], trows_v, sem)
        cp2 = pltpu.async_copy(
            gat_c.at[pl.ds(base * C, 128 * C)], crows_v, sem)
        cp1.wait()
        cp2.wait()

        def body(g, _):
            for bl in range(16):
                i = g * 16 + bl
                t = [trows_v[i, pl.ds(16 * k, 16)] for k in range(4)]
                for c in range(C):
                    r = i * C + c
                    acc = t[0] * crows_v[r, pl.ds(0, 16)]
                    for k in range(1, 4):
                        acc = acc + t[k] * crows_v[r, pl.ds(16 * k, 16)]
                    cums = jnp.cumsum(acc)
                    plsc.store_scatter(out_v, [_I16(r)], cums, mask=lane15)
            return 0

        lax.fori_loop(0, 8, body, 0)
        pltpu.sync_copy(out_v, out_hbm.at[pl.ds(base * C, 128 * C)])


def kernel(target, context, target_table, context_table):
    mesh = plsc.VectorSubcoreMesh(core_axis_name="c", subcore_axis_name="s")
    ctx_flat = context.reshape(N_C).astype(jnp.int32)
    tgt = target.astype(jnp.int32)

    i32 = jnp.int32
    hists = functools.partial(
        pl.kernel, mesh=mesh,
        compiler_params=pltpu.CompilerParams(
            needs_layout_passes=False, use_tc_tiling_on_sc=False),
        out_type=(
            jax.ShapeDtypeStruct((NW, NBLKP), i32),
            jax.ShapeDtypeStruct((NW, NBLKP), i32),
        ),
        scratch_types=(
            [pltpu.VMEM((N_C // NW,), i32)]
            + [pltpu.VMEM((NBLKP,), i32)] * 5
        ),
    )(_hist_body)

    hw_t, hw_c = hists(tgt, ctx_flat)

    binned = functools.partial(
        pl.kernel, mesh=mesh,
        compiler_params=pltpu.CompilerParams(
            needs_layout_passes=False, use_tc_tiling_on_sc=False),
        out_type=(
            jax.ShapeDtypeStruct((N_T + PAD,), i32),
            jax.ShapeDtypeStruct((N_T + PAD,), i32),
            jax.ShapeDtypeStruct((N_C + PAD,), i32),
            jax.ShapeDtypeStruct((N_C + PAD,), i32),
            jax.ShapeDtypeStruct((NBLKP,), i32),
            jax.ShapeDtypeStruct((NBLKP,), i32),
        ),
        scratch_types=[
            pltpu.VMEM((N_C // NW,), i32),
            pltpu.VMEM((NW, NBLKP), i32),
            pltpu.VMEM((NBLKP,), i32),
            pltpu.VMEM((NBLKP,), i32),
            pltpu.VMEM((N_C // NW // 128, 128), i32),
            pltpu.VMEM((N_C // NW // 128, 128), i32),
            pltpu.SemaphoreType.DMA,
        ],
    )(_bin_body)

    pv_t, pp_t, pv_c, pp_c, off_t, off_c = binned(tgt, ctx_flat, hw_t, hw_c)

    swept = functools.partial(
        pl.kernel, mesh=mesh,
        compiler_params=pltpu.CompilerParams(
            needs_layout_passes=False, use_tc_tiling_on_sc=True),
        out_type=(
            jax.ShapeDtypeStruct((N_T + PAD, 128), jnp.float32),
            jax.ShapeDtypeStruct((N_C + PAD, 128), jnp.float32),
        ),
        scratch_types=[
            pltpu.VMEM((D, W), jnp.float32),
            pltpu.VMEM((D, W), jnp.float32),
            pltpu.VMEM((D, 64), jnp.float32),
            pltpu.VMEM((128, 128), jnp.float32),
            pltpu.VMEM((128, 128), jnp.float32),
            pltpu.VMEM((NBLKP,), i32),
            pltpu.VMEM((128,), i32),
            pltpu.VMEM((128,), i32),
            pltpu.VMEM((128,), i32),
            pltpu.VMEM((128,), i32),
            pltpu.VMEM((128,), i32),
            pltpu.VMEM((128,), i32),
            pltpu.SemaphoreType.DMA,
            pltpu.SemaphoreType.DMA,
            pltpu.SemaphoreType.DMA,
            pltpu.SemaphoreType.DMA,
        ],
    )(_sweep_body)

    gat_t, gat_c = swept(target_table.T, context_table.T,
                         pv_t, pp_t, pv_c, pp_c, off_t, off_c)

    dots = functools.partial(
        pl.kernel, mesh=mesh,
        compiler_params=pltpu.CompilerParams(
            needs_layout_passes=False, use_tc_tiling_on_sc=True),
        out_type=jax.ShapeDtypeStruct((N_C,), jnp.float32),
        scratch_types=[
            pltpu.VMEM((128, 128), jnp.float32),
            pltpu.VMEM((128 * C, 128), jnp.float32),
            pltpu.VMEM((128 * C,), jnp.float32),
            pltpu.SemaphoreType.DMA,
        ],
    )(_dots_body)

    out = dots(gat_t, gat_c)
    return out.reshape(B, C)
